# baseline (speedup 1.0000x reference)
import functools

import numpy as np

import jax
import jax.numpy as jnp
from jax.experimental import pallas as pl
from jax.experimental.pallas import tpu as pltpu
from jax.experimental.pallas import tpu_sc as plsc

_NUM_CORES = 2
_NUM_SUBCORES = 16
_NUM_WORKERS = _NUM_CORES * _NUM_SUBCORES
_CHUNK = 128


def _sc_gather_tmajor(table, idx3d, pos3d):
    workers, chunks, cw = idx3d.shape
    n = workers * chunks * cw
    d = table.shape[1]
    rows_per_w = chunks * cw

    mesh = plsc.VectorSubcoreMesh(core_axis_name="c", subcore_axis_name="s")

    @functools.partial(
        pl.kernel,
        out_type=jax.ShapeDtypeStruct((n, d), table.dtype),
        mesh=mesh,
        scratch_types=[
            pltpu.VMEM((chunks, _CHUNK), jnp.int32),
            pltpu.VMEM((chunks, _CHUNK), jnp.int32),
            pltpu.VMEM((rows_per_w, d), table.dtype),
            pltpu.SemaphoreType.DMA,
            pltpu.SemaphoreType.DMA,
        ],
    )
    def gather_kernel(table_hbm, idx_hbm, pos_hbm, out_hbm, idx_v, pos_v,
                      rows_v, gsem, ssem):
        c = jax.lax.axis_index("c")
        s = jax.lax.axis_index("s")
        wid = s * _NUM_CORES + c
        pltpu.sync_copy(idx_hbm.at[wid], idx_v)
        pltpu.sync_copy(pos_hbm.at[wid], pos_v)
        gathers = [
            pltpu.async_copy(
                table_hbm.at[idx_v.at[k]],
                rows_v.at[pl.ds(k * _CHUNK, _CHUNK)],
                gsem,
            )
            for k in range(chunks)
        ]
        for g in gathers:
            g.wait()
        scatters = [
            pltpu.async_copy(
                rows_v.at[pl.ds(k * _CHUNK, _CHUNK)],
                out_hbm.at[pos_v.at[k]],
                ssem,
            )
            for k in range(chunks)
        ]
        for sc in scatters:
            sc.wait()

    return gather_kernel(table, idx3d, pos3d)


def _gru_stack(x3d, k0, r0, b0, k1, r1, b1):
    T, B, E = x3d.shape
    H = r0.shape[0]

    def body(x_ref, k0_ref, r0_ref, b0_ref, k1_ref, r1_ref, b1_ref, y_ref):
        k0v = k0_ref[...].astype(jnp.bfloat16)
        r0v = r0_ref[...].astype(jnp.bfloat16)
        k1v = k1_ref[...].astype(jnp.bfloat16)
        r1v = r1_ref[...].astype(jnp.bfloat16)
        bi0 = b0_ref[0:1, :]
        br0 = b0_ref[1:2, :]
        bi1 = b1_ref[0:1, :]
        br1 = b1_ref[1:2, :]

        def gru_step(h, x_proj, rec, br):
            hp = jnp.dot(h.astype(jnp.bfloat16), rec,
                         preferred_element_type=jnp.float32) + br
            z = jax.nn.sigmoid(x_proj[:, :H] + hp[:, :H])
            r = jax.nn.sigmoid(x_proj[:, H:2 * H] + hp[:, H:2 * H])
            hh = jnp.tanh(x_proj[:, 2 * H:] + r * hp[:, 2 * H:])
            return h + (1.0 - z) * (hh - h)

        h0 = jnp.zeros((B, H), jnp.float32)
        h1 = jnp.zeros((B, H), jnp.float32)
        for t in range(T):
            xp0 = jnp.dot(x_ref[t].astype(jnp.bfloat16), k0v,
                          preferred_element_type=jnp.float32) + bi0
            h0 = gru_step(h0, xp0, r0v, br0)
            xp1 = jnp.dot(h0.astype(jnp.bfloat16), k1v,
                          preferred_element_type=jnp.float32) + bi1
            h1 = gru_step(h1, xp1, r1v, br1)
            y_ref[t] = h1

    return pl.pallas_call(
        body,
        out_shape=jax.ShapeDtypeStruct((T, B, H), jnp.float32),
    )(x3d, k0, r0, b0, k1, r1, b1)


def _proj(y3d, wt, b_col, blk=256):
    t_total, bsz, h = y3d.shape
    v = wt.shape[0]

    def body(y_ref, w_ref, b_ref, o_ref):
        wv = w_ref[...].astype(jnp.bfloat16)
        bv = b_ref[...]
        for t in range(t_total):
            o_ref[t] = jax.lax.dot_general(
                wv, y_ref[t].astype(jnp.bfloat16),
                dimension_numbers=(((1,), (1,)), ((), ())),
                preferred_element_type=jnp.float32,
            ) + bv

    return pl.pallas_call(
        body,
        grid=(bsz // blk,),
        in_specs=[
            pl.BlockSpec((t_total, blk, h), lambda i: (0, i, 0)),
            pl.BlockSpec((v, h), lambda i: (0, 0)),
            pl.BlockSpec((v, 1), lambda i: (0, 0)),
        ],
        out_specs=pl.BlockSpec((t_total, v, blk), lambda i: (0, 0, i)),
        out_shape=jax.ShapeDtypeStruct((t_total, v, bsz), jnp.float32),
        compiler_params=pltpu.CompilerParams(
            dimension_semantics=("arbitrary",),
        ),
    )(y3d, wt, b_col)


def kernel(indices, embed, gru0_kernel, gru0_rec_kernel, gru0_bias,
           gru1_kernel, gru1_rec_kernel, gru1_bias, proj_W, proj_b):
    B, T = indices.shape
    V, E = embed.shape
    H = gru0_rec_kernel.shape[0]

    idx3d = indices.reshape(_NUM_WORKERS, -1, _CHUNK)
    bb, tt = np.meshgrid(np.arange(B), np.arange(T), indexing="ij")
    pos3d = jnp.asarray(
        (tt * B + bb).reshape(_NUM_WORKERS, -1, _CHUNK), dtype=jnp.int32)
    x = _sc_gather_tmajor(embed, idx3d, pos3d).reshape(T, B, E)
    y = _gru_stack(x, gru0_kernel, gru0_rec_kernel, gru0_bias,
                   gru1_kernel, gru1_rec_kernel, gru1_bias)
    out_p = _proj(y, proj_W.T, proj_b.reshape(V, 1))
    return jnp.transpose(out_p, (2, 0, 1))


# speedup vs baseline: 1.1963x; 1.1963x over previous
import functools

import numpy as np

import jax
import jax.numpy as jnp
from jax.experimental import pallas as pl
from jax.experimental.pallas import tpu as pltpu
from jax.experimental.pallas import tpu_sc as plsc

_NUM_CORES = 2
_NUM_SUBCORES = 16
_NUM_WORKERS = _NUM_CORES * _NUM_SUBCORES
_CHUNK = 128


def _sc_gather_tmajor(table, idx3d, pos3d):
    workers, chunks, cw = idx3d.shape
    n = workers * chunks * cw
    d = table.shape[1]
    rows_per_w = chunks * cw

    mesh = plsc.VectorSubcoreMesh(core_axis_name="c", subcore_axis_name="s")

    @functools.partial(
        pl.kernel,
        out_type=jax.ShapeDtypeStruct((n, d), table.dtype),
        mesh=mesh,
        scratch_types=[
            pltpu.VMEM((chunks, _CHUNK), jnp.int32),
            pltpu.VMEM((chunks, _CHUNK), jnp.int32),
            pltpu.VMEM((rows_per_w, d), table.dtype),
            pltpu.SemaphoreType.DMA,
            pltpu.SemaphoreType.DMA,
        ],
    )
    def gather_kernel(table_hbm, idx_hbm, pos_hbm, out_hbm, idx_v, pos_v,
                      rows_v, gsem, ssem):
        c = jax.lax.axis_index("c")
        s = jax.lax.axis_index("s")
        wid = s * _NUM_CORES + c
        pltpu.sync_copy(idx_hbm.at[wid], idx_v)
        pltpu.sync_copy(pos_hbm.at[wid], pos_v)
        gathers = [
            pltpu.async_copy(
                table_hbm.at[idx_v.at[k]],
                rows_v.at[pl.ds(k * _CHUNK, _CHUNK)],
                gsem,
            )
            for k in range(chunks)
        ]
        for g in gathers:
            g.wait()
        scatters = [
            pltpu.async_copy(
                rows_v.at[pl.ds(k * _CHUNK, _CHUNK)],
                out_hbm.at[pos_v.at[k]],
                ssem,
            )
            for k in range(chunks)
        ]
        for sc in scatters:
            sc.wait()

    return gather_kernel(table, idx3d, pos3d)


def _gru_proj(x3d, k0, r0, b0, k1, r1, b1, wt, b_col):
    T, B, E = x3d.shape
    H = r0.shape[0]
    v = wt.shape[0]

    def body(x_ref, k0_ref, r0_ref, b0_ref, k1_ref, r1_ref, b1_ref,
             wt_ref, b_ref, o_ref, h0_s, h1_s):
        t = pl.program_id(0)

        @pl.when(t == 0)
        def _init():
            h0_s[...] = jnp.zeros_like(h0_s)
            h1_s[...] = jnp.zeros_like(h1_s)

        k0v = k0_ref[...].astype(jnp.bfloat16)
        r0v = r0_ref[...].astype(jnp.bfloat16)
        k1v = k1_ref[...].astype(jnp.bfloat16)
        r1v = r1_ref[...].astype(jnp.bfloat16)
        bi0 = b0_ref[0:1, :]
        br0 = b0_ref[1:2, :]
        bi1 = b1_ref[0:1, :]
        br1 = b1_ref[1:2, :]

        def gru_step(h, x_proj, rec, br):
            hp = jnp.dot(h.astype(jnp.bfloat16), rec,
                         preferred_element_type=jnp.float32) + br
            z = jax.nn.sigmoid(x_proj[:, :H] + hp[:, :H])
            r = jax.nn.sigmoid(x_proj[:, H:2 * H] + hp[:, H:2 * H])
            hh = jnp.tanh(x_proj[:, 2 * H:] + r * hp[:, 2 * H:])
            return h + (1.0 - z) * (hh - h)

        xp0 = jnp.dot(x_ref[0].astype(jnp.bfloat16), k0v,
                      preferred_element_type=jnp.float32) + bi0
        h0 = gru_step(h0_s[...], xp0, r0v, br0)
        h0_s[...] = h0
        xp1 = jnp.dot(h0.astype(jnp.bfloat16), k1v,
                      preferred_element_type=jnp.float32) + bi1
        h1 = gru_step(h1_s[...], xp1, r1v, br1)
        h1_s[...] = h1
        o_ref[0] = jax.lax.dot_general(
            wt_ref[...].astype(jnp.bfloat16), h1.astype(jnp.bfloat16),
            dimension_numbers=(((1,), (1,)), ((), ())),
            preferred_element_type=jnp.float32,
        ) + b_ref[...]

    return pl.pallas_call(
        body,
        grid=(T,),
        in_specs=[
            pl.BlockSpec((1, B, E), lambda t: (t, 0, 0)),
            pl.BlockSpec(k0.shape, lambda t: (0, 0)),
            pl.BlockSpec(r0.shape, lambda t: (0, 0)),
            pl.BlockSpec(b0.shape, lambda t: (0, 0)),
            pl.BlockSpec(k1.shape, lambda t: (0, 0)),
            pl.BlockSpec(r1.shape, lambda t: (0, 0)),
            pl.BlockSpec(b1.shape, lambda t: (0, 0)),
            pl.BlockSpec(wt.shape, lambda t: (0, 0)),
            pl.BlockSpec((v, 1), lambda t: (0, 0)),
        ],
        out_specs=pl.BlockSpec((1, v, B), lambda t: (t, 0, 0)),
        out_shape=jax.ShapeDtypeStruct((T, v, B), jnp.float32),
        scratch_shapes=[
            pltpu.VMEM((B, H), jnp.float32),
            pltpu.VMEM((B, H), jnp.float32),
        ],
        compiler_params=pltpu.CompilerParams(
            dimension_semantics=("arbitrary",),
        ),
    )(x3d, k0, r0, b0, k1, r1, b1, wt, b_col)


def kernel(indices, embed, gru0_kernel, gru0_rec_kernel, gru0_bias,
           gru1_kernel, gru1_rec_kernel, gru1_bias, proj_W, proj_b):
    B, T = indices.shape
    V, E = embed.shape
    H = gru0_rec_kernel.shape[0]

    idx3d = indices.reshape(_NUM_WORKERS, -1, _CHUNK)
    bb, tt = np.meshgrid(np.arange(B), np.arange(T), indexing="ij")
    pos3d = jnp.asarray(
        (tt * B + bb).reshape(_NUM_WORKERS, -1, _CHUNK), dtype=jnp.int32)
    x = _sc_gather_tmajor(embed, idx3d, pos3d).reshape(T, B, E)
    out_p = _gru_proj(x, gru0_kernel, gru0_rec_kernel, gru0_bias,
                      gru1_kernel, gru1_rec_kernel, gru1_bias,
                      proj_W.T, proj_b.reshape(V, 1))
    return jnp.transpose(out_p, (2, 0, 1))


# speedup vs baseline: 1.1971x; 1.0006x over previous
import functools

import numpy as np

import jax
import jax.numpy as jnp
from jax.experimental import pallas as pl
from jax.experimental.pallas import tpu as pltpu
from jax.experimental.pallas import tpu_sc as plsc

_NUM_CORES = 2
_NUM_SUBCORES = 16
_NUM_WORKERS = _NUM_CORES * _NUM_SUBCORES
_CHUNK = 128


def _sc_gather_tmajor(table, idx3d, pos3d):
    workers, chunks, cw = idx3d.shape
    n = workers * chunks * cw
    d = table.shape[1]
    rows_per_w = chunks * cw

    mesh = plsc.VectorSubcoreMesh(core_axis_name="c", subcore_axis_name="s")

    @functools.partial(
        pl.kernel,
        out_type=jax.ShapeDtypeStruct((n, d), table.dtype),
        mesh=mesh,
        scratch_types=[
            pltpu.VMEM((chunks, _CHUNK), jnp.int32),
            pltpu.VMEM((chunks, _CHUNK), jnp.int32),
            pltpu.VMEM((rows_per_w, d), table.dtype),
            pltpu.SemaphoreType.DMA,
            pltpu.SemaphoreType.DMA,
        ],
    )
    def gather_kernel(table_hbm, idx_hbm, pos_hbm, out_hbm, idx_v, pos_v,
                      rows_v, gsem, ssem):
        c = jax.lax.axis_index("c")
        s = jax.lax.axis_index("s")
        wid = s * _NUM_CORES + c
        pltpu.sync_copy(idx_hbm.at[wid], idx_v)
        pltpu.sync_copy(pos_hbm.at[wid], pos_v)
        gathers = [
            pltpu.async_copy(
                table_hbm.at[idx_v.at[k]],
                rows_v.at[pl.ds(k * _CHUNK, _CHUNK)],
                gsem,
            )
            for k in range(chunks)
        ]
        for g in gathers:
            g.wait()
        scatters = [
            pltpu.async_copy(
                rows_v.at[pl.ds(k * _CHUNK, _CHUNK)],
                out_hbm.at[pos_v.at[k]],
                ssem,
            )
            for k in range(chunks)
        ]
        for sc in scatters:
            sc.wait()

    return gather_kernel(table, idx3d, pos3d)


def _gru_proj(x3d, k0, r0, b0, k1, r1, b1, wt, b_col):
    T, B, E = x3d.shape
    H = r0.shape[0]
    v = wt.shape[0]

    def body(x_ref, k0_ref, r0_ref, b0_ref, k1_ref, r1_ref, b1_ref,
             wt_ref, b_ref, o_ref, h0_s, h1_s):
        t = pl.program_id(0)

        @pl.when(t == 0)
        def _init():
            h0_s[...] = jnp.zeros_like(h0_s)
            h1_s[...] = jnp.zeros_like(h1_s)

        k0v = k0_ref[...].astype(jnp.bfloat16)
        r0v = r0_ref[...].astype(jnp.bfloat16)
        k1v = k1_ref[...].astype(jnp.bfloat16)
        r1v = r1_ref[...].astype(jnp.bfloat16)
        bi0 = b0_ref[0:1, :]
        br0 = b0_ref[1:2, :]
        bi1 = b1_ref[0:1, :]
        br1 = b1_ref[1:2, :]

        def gru_step(h, x_proj, rec, br):
            hp = jnp.dot(h.astype(jnp.bfloat16), rec,
                         preferred_element_type=jnp.float32) + br
            z = jax.nn.sigmoid(x_proj[:, :H] + hp[:, :H])
            r = jax.nn.sigmoid(x_proj[:, H:2 * H] + hp[:, H:2 * H])
            hh = jnp.tanh(x_proj[:, 2 * H:] + r * hp[:, 2 * H:])
            return h + (1.0 - z) * (hh - h)

        xp0 = jnp.dot(x_ref[0].astype(jnp.bfloat16), k0v,
                      preferred_element_type=jnp.float32) + bi0
        h0 = gru_step(h0_s[...], xp0, r0v, br0)
        h0_s[...] = h0
        xp1 = jnp.dot(h0.astype(jnp.bfloat16), k1v,
                      preferred_element_type=jnp.float32) + bi1
        h1 = gru_step(h1_s[...], xp1, r1v, br1)
        h1_s[...] = h1
        o_ref[0] = jax.lax.dot_general(
            wt_ref[...].astype(jnp.bfloat16), h1.astype(jnp.bfloat16),
            dimension_numbers=(((1,), (1,)), ((), ())),
            preferred_element_type=jnp.float32,
        ) + b_ref[...]

    return pl.pallas_call(
        body,
        grid=(T,),
        in_specs=[
            pl.BlockSpec((1, B, E), lambda t: (t, 0, 0)),
            pl.BlockSpec(k0.shape, lambda t: (0, 0)),
            pl.BlockSpec(r0.shape, lambda t: (0, 0)),
            pl.BlockSpec(b0.shape, lambda t: (0, 0)),
            pl.BlockSpec(k1.shape, lambda t: (0, 0)),
            pl.BlockSpec(r1.shape, lambda t: (0, 0)),
            pl.BlockSpec(b1.shape, lambda t: (0, 0)),
            pl.BlockSpec(wt.shape, lambda t: (0, 0)),
            pl.BlockSpec((v, 1), lambda t: (0, 0)),
        ],
        out_specs=pl.BlockSpec((1, v, B), lambda t: (t, 0, 0)),
        out_shape=jax.ShapeDtypeStruct((T, v, B), jnp.float32),
        scratch_shapes=[
            pltpu.VMEM((B, H), jnp.float32),
            pltpu.VMEM((B, H), jnp.float32),
        ],
        compiler_params=pltpu.CompilerParams(
            dimension_semantics=("arbitrary",),
        ),
    )(x3d, k0, r0, b0, k1, r1, b1, wt, b_col)


def kernel(indices, embed, gru0_kernel, gru0_rec_kernel, gru0_bias,
           gru1_kernel, gru1_rec_kernel, gru1_bias, proj_W, proj_b):
    B, T = indices.shape
    V, E = embed.shape

    idx3d = indices.reshape(_NUM_WORKERS, -1, _CHUNK)
    bb, tt = np.meshgrid(np.arange(B), np.arange(T), indexing="ij")
    pos3d = jnp.asarray(
        (tt * B + bb).reshape(_NUM_WORKERS, -1, _CHUNK), dtype=jnp.int32)
    x = _sc_gather_tmajor(embed, idx3d, pos3d).reshape(T, B, E)
    out_p = _gru_proj(x, gru0_kernel, gru0_rec_kernel, gru0_bias,
                      gru1_kernel, gru1_rec_kernel, gru1_bias,
                      proj_W.T, proj_b.reshape(V, 1))
    return jnp.transpose(out_p, (2, 0, 1))
